# baseline (speedup 1.0000x reference)
import functools

import jax
import jax.numpy as jnp
from jax import lax
from jax.experimental import pallas as pl
from jax.experimental.pallas import tpu as pltpu
from jax.experimental.pallas import tpu_sc as plsc

_VOCAB = 1000000
_D = 32
_LINE = 128
_REP = _LINE // _D
_REP_BLK = 8000
_POST_BLK = 8192

_NC, _NS = 2, 16
_NW = _NC * _NS
_IDX_W = 128
_K = 4
_CHUNK = _IDX_W * _K


def _rep_body(w_ref, o_ref):
    x = w_ref[...]
    o_ref[...] = jnp.concatenate([x] * _REP, axis=-1)


def _replicate_table(W):
    return pl.pallas_call(
        _rep_body,
        grid=(_VOCAB // _REP_BLK,),
        in_specs=[pl.BlockSpec((_REP_BLK, _D), lambda i: (i, 0))],
        out_specs=pl.BlockSpec((_REP_BLK, _LINE), lambda i: (i, 0)),
        out_shape=jax.ShapeDtypeStruct((_VOCAB, _LINE), jnp.float32),
    )(W)


def _gather_lines(Wr, ids2d):
    n_rows = ids2d.shape[0] * _IDX_W
    rows_per_w = n_rows // _NW
    chunks_per_w = rows_per_w // _CHUNK
    idx_rows_per_w = ids2d.shape[0] // _NW

    mesh = plsc.VectorSubcoreMesh(core_axis_name="c", subcore_axis_name="s")

    @functools.partial(
        pl.kernel,
        out_type=jax.ShapeDtypeStruct((n_rows, _LINE), jnp.float32),
        mesh=mesh,
        scratch_types=[
            pltpu.VMEM((_K, _IDX_W), jnp.int32),
            pltpu.VMEM((_CHUNK, _LINE), jnp.float32),
            pltpu.SemaphoreType.DMA,
        ],
    )
    def k(w_hbm, i_hbm, o_hbm, idx_v, rows_v, sem):
        wid = lax.axis_index("s") * _NC + lax.axis_index("c")
        idx_row0 = wid * idx_rows_per_w
        out0 = wid * rows_per_w

        @pl.loop(0, chunks_per_w)
        def _(c):
            pltpu.sync_copy(i_hbm.at[pl.ds(idx_row0 + c * _K, _K)], idx_v)
            copies = []
            for j in range(_K):
                copies.append(
                    pltpu.async_copy(
                        w_hbm.at[idx_v.at[j]],
                        rows_v.at[pl.ds(j * _IDX_W, _IDX_W)],
                        sem,
                    )
                )
            for cp in copies:
                cp.wait()
            pltpu.sync_copy(rows_v, o_hbm.at[pl.ds(out0 + c * _CHUNK, _CHUNK)])

    return k(Wr, ids2d)


def _post_body(g_ref, o_ref):
    x = g_ref[...][:, :_D]
    s = jnp.sum(x * x, axis=-1, keepdims=True)
    o_ref[...] = x / jnp.maximum(jnp.sqrt(s), 1e-12)


def _extract_normalize(G):
    n = G.shape[0]
    return pl.pallas_call(
        _post_body,
        grid=(n // _POST_BLK,),
        in_specs=[pl.BlockSpec((_POST_BLK, _LINE), lambda i: (i, 0))],
        out_specs=pl.BlockSpec((_POST_BLK, _D), lambda i: (i, 0)),
        out_shape=jax.ShapeDtypeStruct((n, _D), jnp.float32),
    )(G)


def kernel(species_ids, W):
    ids = jnp.clip(species_ids.astype(jnp.int32), 0, _VOCAB - 1)
    Wr = _replicate_table(W)
    lines = _gather_lines(Wr, ids.reshape(-1, _IDX_W))
    out = _extract_normalize(lines)
    return out.reshape(ids.shape + (_D,))


# speedup vs baseline: 1.1063x; 1.1063x over previous
import functools

import jax
import jax.numpy as jnp
from jax import lax
from jax.experimental import pallas as pl
from jax.experimental.pallas import tpu as pltpu
from jax.experimental.pallas import tpu_sc as plsc

_VOCAB = 1000000
_D = 32
_POST_BLK = 8192

_NC, _NS = 2, 16
_NW = _NC * _NS
_IDX_W = 128
_K = 8
_CHUNK = _IDX_W * _K


def _gather_rows(W, ids2d):
    n_rows = ids2d.shape[0] * _IDX_W
    rows_per_w = n_rows // _NW
    chunks_per_w = rows_per_w // _CHUNK
    idx_rows_per_w = ids2d.shape[0] // _NW

    mesh = plsc.VectorSubcoreMesh(core_axis_name="c", subcore_axis_name="s")

    @functools.partial(
        pl.kernel,
        out_type=jax.ShapeDtypeStruct((n_rows, _D), jnp.float32),
        mesh=mesh,
        scratch_types=[
            pltpu.VMEM((_K, _IDX_W), jnp.int32),
            pltpu.VMEM((_CHUNK, _D), jnp.float32),
            pltpu.SemaphoreType.DMA,
        ],
        compiler_params=pltpu.CompilerParams(use_tc_tiling_on_sc=False),
    )
    def k(w_hbm, i_hbm, o_hbm, idx_v, rows_v, sem):
        wid = lax.axis_index("s") * _NC + lax.axis_index("c")
        idx_row0 = wid * idx_rows_per_w
        out0 = wid * rows_per_w

        @pl.loop(0, chunks_per_w)
        def _(c):
            pltpu.sync_copy(i_hbm.at[pl.ds(idx_row0 + c * _K, _K)], idx_v)
            copies = []
            for j in range(_K):
                copies.append(
                    pltpu.async_copy(
                        w_hbm.at[idx_v.at[j]],
                        rows_v.at[pl.ds(j * _IDX_W, _IDX_W)],
                        sem,
                    )
                )
            for cp in copies:
                cp.wait()
            pltpu.sync_copy(rows_v, o_hbm.at[pl.ds(out0 + c * _CHUNK, _CHUNK)])

    return k(W, ids2d)


def _post_body(g_ref, o_ref):
    x = g_ref[...]
    s = jnp.sum(x * x, axis=-1, keepdims=True)
    o_ref[...] = x / jnp.maximum(jnp.sqrt(s), 1e-12)


def _normalize(G):
    n = G.shape[0]
    return pl.pallas_call(
        _post_body,
        grid=(n // _POST_BLK,),
        in_specs=[pl.BlockSpec((_POST_BLK, _D), lambda i: (i, 0))],
        out_specs=pl.BlockSpec((_POST_BLK, _D), lambda i: (i, 0)),
        out_shape=jax.ShapeDtypeStruct((n, _D), jnp.float32),
    )(G)


def kernel(species_ids, W):
    ids = jnp.clip(species_ids.astype(jnp.int32), 0, _VOCAB - 1)
    rows = _gather_rows(W, ids.reshape(-1, _IDX_W))
    out = _normalize(rows)
    return out.reshape(ids.shape + (_D,))


# speedup vs baseline: 1.7840x; 1.6125x over previous
import functools

import jax
import jax.numpy as jnp
from jax import lax
from jax.experimental import pallas as pl
from jax.experimental.pallas import tpu as pltpu
from jax.experimental.pallas import tpu_sc as plsc

_VOCAB = 1000000
_D = 32
_POST_BLK = 8192

_NC, _NS = 2, 16
_NW = _NC * _NS
_IDX_W = 128
_K = 8
_CHUNK = _IDX_W * _K


def _gather_rows(W, ids2d):
    n_rows = ids2d.shape[0] * _IDX_W
    rows_per_w = n_rows // _NW
    chunks_per_w = rows_per_w // _CHUNK
    idx_rows_per_w = ids2d.shape[0] // _NW

    mesh = plsc.VectorSubcoreMesh(core_axis_name="c", subcore_axis_name="s")

    @functools.partial(
        pl.kernel,
        out_type=jax.ShapeDtypeStruct((n_rows, _D), jnp.float32),
        mesh=mesh,
        scratch_types=[
            pltpu.VMEM((_K, _IDX_W), jnp.int32),
            pltpu.VMEM((_CHUNK, _D), jnp.float32),
            pltpu.SemaphoreType.DMA,
        ],
        compiler_params=pltpu.CompilerParams(use_tc_tiling_on_sc=False),
    )
    def k(w_hbm, i_hbm, o_hbm, idx_v, rows_v, sem):
        wid = lax.axis_index("s") * _NC + lax.axis_index("c")
        idx_row0 = wid * idx_rows_per_w
        out0 = wid * rows_per_w

        @pl.loop(0, chunks_per_w)
        def _(c):
            pltpu.sync_copy(i_hbm.at[pl.ds(idx_row0 + c * _K, _K)], idx_v)
            copies = []
            for j in range(_K):
                copies.append(
                    pltpu.async_copy(
                        w_hbm.at[idx_v.at[j]],
                        rows_v.at[pl.ds(j * _IDX_W, _IDX_W)],
                        sem,
                    )
                )
            for cp in copies:
                cp.wait()
            pltpu.sync_copy(rows_v, o_hbm.at[pl.ds(out0 + c * _CHUNK, _CHUNK)])

    return k(W, ids2d)


_BB = 64
_G = 128 // _D


def _post_body(g_ref, o_ref):
    x = g_ref[...]
    li = lax.broadcasted_iota(jnp.int32, (128, 128), 0) // _D
    lj = lax.broadcasted_iota(jnp.int32, (128, 128), 1) // _D
    m = (li == lj).astype(jnp.float32)
    s = jax.lax.dot(x * x, m, precision=jax.lax.Precision.HIGHEST)
    o_ref[...] = x / jnp.maximum(jnp.sqrt(s), 1e-12)


def _normalize(G):
    n = G.shape[0]
    blk = _BB * 50 // _G
    return pl.pallas_call(
        _post_body,
        grid=(n // blk,),
        in_specs=[pl.BlockSpec((blk, 128), lambda i: (i, 0))],
        out_specs=pl.BlockSpec((blk, 128), lambda i: (i, 0)),
        out_shape=jax.ShapeDtypeStruct((n, 128), jnp.float32),
    )(G)


def kernel(species_ids, W):
    ids = jnp.clip(species_ids.astype(jnp.int32), 0, _VOCAB - 1)
    rows = _gather_rows(W, ids.reshape(-1, _IDX_W))
    lines = rows.reshape(-1, 128)
    normed = _normalize(lines)
    return normed.reshape(-1, _D).reshape(ids.shape + (_D,))


# speedup vs baseline: 1.8255x; 1.0233x over previous
import functools

import jax
import jax.numpy as jnp
from jax import lax
from jax.experimental import pallas as pl
from jax.experimental.pallas import tpu as pltpu
from jax.experimental.pallas import tpu_sc as plsc

_VOCAB = 1000000
_D = 32
_POST_BLK = 8192

_NC, _NS = 2, 16
_NW = _NC * _NS
_IDX_W = 128
_K = 8
_CHUNK = _IDX_W * _K


def _gather_rows(W, ids2d):
    n_rows = ids2d.shape[0] * _IDX_W
    rows_per_w = n_rows // _NW
    chunks_per_w = rows_per_w // _CHUNK
    idx_rows_per_w = ids2d.shape[0] // _NW

    mesh = plsc.VectorSubcoreMesh(core_axis_name="c", subcore_axis_name="s")

    @functools.partial(
        pl.kernel,
        out_type=jax.ShapeDtypeStruct((n_rows, _D), jnp.float32),
        mesh=mesh,
        scratch_types=[
            pltpu.VMEM((_K, _IDX_W), jnp.int32),
            pltpu.VMEM((_CHUNK, _D), jnp.float32),
            pltpu.SemaphoreType.DMA,
        ],
        compiler_params=pltpu.CompilerParams(use_tc_tiling_on_sc=False),
    )
    def k(w_hbm, i_hbm, o_hbm, idx_v, rows_v, sem):
        wid = lax.axis_index("s") * _NC + lax.axis_index("c")
        idx_row0 = wid * idx_rows_per_w
        out0 = wid * rows_per_w

        @pl.loop(0, chunks_per_w)
        def _(c):
            pltpu.sync_copy(i_hbm.at[pl.ds(idx_row0 + c * _K, _K)], idx_v)
            copies = []
            for j in range(_K):
                copies.append(
                    pltpu.async_copy(
                        w_hbm.at[idx_v.at[j]],
                        rows_v.at[pl.ds(j * _IDX_W, _IDX_W)],
                        sem,
                    )
                )
            for cp in copies:
                cp.wait()
            pltpu.sync_copy(rows_v, o_hbm.at[pl.ds(out0 + c * _CHUNK, _CHUNK)])

    return k(W, ids2d)


_BB = 64
_G = 128 // _D


def _post_body(g_ref, o_ref):
    x = g_ref[...]
    li = lax.broadcasted_iota(jnp.int32, (128, 128), 0) // _D
    lj = lax.broadcasted_iota(jnp.int32, (128, 128), 1) // _D
    m = (li == lj).astype(jnp.float32)
    s = jax.lax.dot(x * x, m)
    o_ref[...] = x / jnp.maximum(jnp.sqrt(s), 1e-12)


def _normalize(G):
    n = G.shape[0]
    blk = _BB * 50 // _G
    return pl.pallas_call(
        _post_body,
        grid=(n // blk,),
        in_specs=[pl.BlockSpec((blk, 128), lambda i: (i, 0))],
        out_specs=pl.BlockSpec((blk, 128), lambda i: (i, 0)),
        out_shape=jax.ShapeDtypeStruct((n, 128), jnp.float32),
    )(G)


def kernel(species_ids, W):
    ids = jnp.clip(species_ids.astype(jnp.int32), 0, _VOCAB - 1)
    rows = _gather_rows(W, ids.reshape(-1, _IDX_W))
    lines = rows.reshape(-1, 128)
    normed = _normalize(lines)
    return normed.reshape(-1, _D).reshape(ids.shape + (_D,))
